# baseline (speedup 1.0000x reference)
import jax
import jax.numpy as jnp
from jax.experimental import pallas as pl
from jax.experimental.pallas import tpu as pltpu

HIDDEN_DIM = 4096
NUM_EXPERTS = 64
PRED_HIDDEN = 256
TILE_M = 256


def _mlp_kernel(xa_ref, xb_ref, w1t_ref, b1_ref, w2t_ref, b2_ref, o_ref):
    w1t = w1t_ref[...]
    w2t = w2t_ref[...]
    b1 = b1_ref[...]
    b2 = b2_ref[...]
    for k, x_ref in enumerate((xa_ref, xb_ref)):
        xb = x_ref[...].astype(jnp.bfloat16)
        h = jnp.dot(xb, w1t, preferred_element_type=jnp.float32)
        h = jnp.maximum(h + b1, 0.0).astype(jnp.bfloat16)
        o_ref[k * TILE_M:(k + 1) * TILE_M, :] = (
            jnp.dot(h, w2t, preferred_element_type=jnp.float32) + b2
        )


def kernel(x, W1, b1, W2, b2, expert_bias):
    orig_shape = x.shape[:-1]
    x2 = x.reshape(-1, HIDDEN_DIM)
    m = x2.shape[0]
    w1t = W1.T.astype(jnp.bfloat16)
    w2t = W2.T.astype(jnp.bfloat16)
    b1r = b1.reshape(1, PRED_HIDDEN)
    b2r = (b2 + expert_bias).reshape(1, NUM_EXPERTS)

    grid = (m // (2 * TILE_M),)
    out = pl.pallas_call(
        _mlp_kernel,
        grid=grid,
        in_specs=[
            pl.BlockSpec((TILE_M, HIDDEN_DIM), lambda i: (2 * i, 0)),
            pl.BlockSpec((TILE_M, HIDDEN_DIM), lambda i: (2 * i + 1, 0)),
            pl.BlockSpec((HIDDEN_DIM, PRED_HIDDEN), lambda i: (0, 0)),
            pl.BlockSpec((1, PRED_HIDDEN), lambda i: (0, 0)),
            pl.BlockSpec((PRED_HIDDEN, NUM_EXPERTS), lambda i: (0, 0)),
            pl.BlockSpec((1, NUM_EXPERTS), lambda i: (0, 0)),
        ],
        out_specs=pl.BlockSpec((2 * TILE_M, NUM_EXPERTS), lambda i: (i, 0)),
        out_shape=jax.ShapeDtypeStruct((m, NUM_EXPERTS), jnp.float32),
        compiler_params=pltpu.CompilerParams(
            dimension_semantics=("parallel",),
        ),
    )(x2, x2, w1t, b1r, w2t, b2r)
    return out.reshape(*orig_shape, NUM_EXPERTS)


# speedup vs baseline: 1.1712x; 1.1712x over previous
import jax
import jax.numpy as jnp
from jax.experimental import pallas as pl
from jax.experimental.pallas import tpu as pltpu

HIDDEN_DIM = 4096
NUM_EXPERTS = 64
PRED_HIDDEN = 256
TILE_M = 1024


def _mlp_kernel(x_ref, w1t_ref, b1_ref, w2t_ref, b2_ref, o_ref):
    xb = x_ref[...].astype(jnp.bfloat16)
    h = jnp.dot(xb, w1t_ref[...], preferred_element_type=jnp.float32)
    h = jnp.maximum(h + b1_ref[...], 0.0).astype(jnp.bfloat16)
    o_ref[...] = (
        jnp.dot(h, w2t_ref[...], preferred_element_type=jnp.float32) + b2_ref[...]
    )


def kernel(x, W1, b1, W2, b2, expert_bias):
    orig_shape = x.shape[:-1]
    x2 = x.reshape(-1, HIDDEN_DIM)
    m = x2.shape[0]
    w1t = W1.T.astype(jnp.bfloat16)
    w2t = W2.T.astype(jnp.bfloat16)
    b1r = b1.reshape(1, PRED_HIDDEN)
    b2r = (b2 + expert_bias).reshape(1, NUM_EXPERTS)

    grid = (m // TILE_M,)
    out = pl.pallas_call(
        _mlp_kernel,
        grid=grid,
        in_specs=[
            pl.BlockSpec((TILE_M, HIDDEN_DIM), lambda i: (i, 0)),
            pl.BlockSpec((HIDDEN_DIM, PRED_HIDDEN), lambda i: (0, 0)),
            pl.BlockSpec((1, PRED_HIDDEN), lambda i: (0, 0)),
            pl.BlockSpec((PRED_HIDDEN, NUM_EXPERTS), lambda i: (0, 0)),
            pl.BlockSpec((1, NUM_EXPERTS), lambda i: (0, 0)),
        ],
        out_specs=pl.BlockSpec((TILE_M, NUM_EXPERTS), lambda i: (i, 0)),
        out_shape=jax.ShapeDtypeStruct((m, NUM_EXPERTS), jnp.float32),
        compiler_params=pltpu.CompilerParams(
            dimension_semantics=("parallel",),
        ),
    )(x2, w1t, b1r, w2t, b2r)
    return out.reshape(*orig_shape, NUM_EXPERTS)
